# baseline (speedup 1.0000x reference)
import functools

import jax
import jax.numpy as jnp
from jax import lax
from jax.experimental import pallas as pl
from jax.experimental.pallas import tpu as pltpu
from jax.experimental.pallas import tpu_sc as plsc

NC = 2
NS = 16
NW = NC * NS

BBLK = 128
NBUF = 4


@functools.lru_cache(maxsize=None)
def _build_gather(vocab, l_seq, batch):
    nblk = batch // BBLK // NW
    bspan = nblk * BBLK
    nblocks = l_seq * nblk
    assert nblocks % NBUF == 0

    mesh = plsc.VectorSubcoreMesh(core_axis_name="c", subcore_axis_name="s")

    @functools.partial(
        pl.kernel,
        out_type=jax.ShapeDtypeStruct((l_seq, batch, 128), jnp.float32),
        mesh=mesh,
        scratch_types=[
            pltpu.VMEM((l_seq, bspan), jnp.int32),
            pltpu.VMEM((NBUF, BBLK, 128), jnp.float32),
            [pltpu.SemaphoreType.DMA] * NBUF,
            [pltpu.SemaphoreType.DMA] * NBUF,
        ],
        compiler_params=pltpu.CompilerParams(use_tc_tiling_on_sc=True,
                                             needs_layout_passes=False),
    )
    def k1(t128_hbm, xt_hbm, out_hbm, idx_v, g_v, gsems, wsems):
        wid = lax.axis_index("s") * NC + lax.axis_index("c")
        b0 = wid * bspan

        pltpu.sync_copy(xt_hbm.at[:, pl.ds(b0, bspan)], idx_v)

        def fire_gather(p, i):
            l, k = i // nblk, i % nblk
            pltpu.async_copy(
                t128_hbm.at[idx_v.at[l, pl.ds(k * BBLK, BBLK)]],
                g_v.at[p], gsems[p])

        def wait_gather(p):
            pltpu.make_async_copy(
                t128_hbm.at[idx_v.at[0, pl.ds(0, BBLK)]],
                g_v.at[p], gsems[p]).wait()

        def fire_write(p, i):
            l, k = i // nblk, i % nblk
            pltpu.async_copy(
                g_v.at[p],
                out_hbm.at[l, pl.ds(b0 + k * BBLK, BBLK), :], wsems[p])

        def wait_write(p):
            pltpu.make_async_copy(
                g_v.at[p],
                out_hbm.at[0, pl.ds(0, BBLK), :], wsems[p]).wait()

        def outer(c, carry):
            for p in range(NBUF):
                @pl.when(c > 0)
                def _(p=p):
                    wait_write(p)

                fire_gather(p, c * NBUF + p)
            for p in range(NBUF):
                wait_gather(p)
                fire_write(p, c * NBUF + p)
            return carry

        lax.fori_loop(0, nblocks // NBUF, outer, 0, unroll=False)
        for p in range(NBUF):
            wait_write(p)

    return k1


@functools.lru_cache(maxsize=None)
def _build_transpose(emb, l_seq, batch):
    tblk = 1024

    def k2(in_ref, out_ref):
        rows = lax.broadcasted_iota(jnp.int32, (emb, emb), 0)
        cols = lax.broadcasted_iota(jnp.int32, (emb, emb), 1)
        eye = jnp.where(rows == cols, 1.0, 0.0).astype(jnp.float32)
        x = in_ref[0, :, :emb]
        out_ref[0] = lax.dot_general(
            eye, x, (((1,), (1,)), ((), ())),
            precision=lax.Precision.HIGHEST,
            preferred_element_type=jnp.float32)

    return pl.pallas_call(
        k2,
        grid=(l_seq, batch // tblk),
        in_specs=[pl.BlockSpec((1, tblk, 128), lambda l, b: (l, b, 0))],
        out_specs=pl.BlockSpec((1, emb, tblk), lambda l, b: (l, 0, b)),
        out_shape=jax.ShapeDtypeStruct((l_seq, emb, batch), jnp.float32),
    )


def kernel(x, TokenEmbeddings):
    batch, l_seq = x.shape
    vocab, emb = TokenEmbeddings.shape
    t128 = jnp.pad(TokenEmbeddings, ((0, 0), (0, 128 - emb)))
    xt = x.T.astype(jnp.int32)
    mid = _build_gather(vocab, l_seq, batch)(t128, xt)
    out3 = _build_transpose(emb, l_seq, batch)(mid)
    return jnp.transpose(out3, (2, 0, 1))


# speedup vs baseline: 1.1197x; 1.1197x over previous
import functools

import jax
import jax.numpy as jnp
from jax import lax
from jax.experimental import pallas as pl
from jax.experimental.pallas import tpu as pltpu
from jax.experimental.pallas import tpu_sc as plsc

NC = 2
NS = 16
NW = NC * NS

BBLK = 128
NBUF = 4


@functools.lru_cache(maxsize=None)
def _build_gather(vocab, l_seq, batch):
    nblk = batch // BBLK // NW
    bspan = nblk * BBLK
    nblocks = l_seq * nblk
    assert nblocks % NBUF == 0

    mesh = plsc.VectorSubcoreMesh(core_axis_name="c", subcore_axis_name="s")

    @functools.partial(
        pl.kernel,
        out_type=jax.ShapeDtypeStruct((l_seq, batch, 128), jnp.float32),
        mesh=mesh,
        scratch_types=[
            pltpu.VMEM((l_seq, bspan), jnp.int32),
            pltpu.VMEM((NBUF, BBLK, 128), jnp.float32),
            [pltpu.SemaphoreType.DMA] * NBUF,
            [pltpu.SemaphoreType.DMA] * NBUF,
        ],
        compiler_params=pltpu.CompilerParams(use_tc_tiling_on_sc=True,
                                             needs_layout_passes=False),
    )
    def k1(t128_hbm, xt_hbm, out_hbm, idx_v, g_v, gsems, wsems):
        wid = lax.axis_index("s") * NC + lax.axis_index("c")
        b0 = wid * bspan

        pltpu.sync_copy(xt_hbm.at[:, pl.ds(b0, bspan)], idx_v)

        def fire_gather(p, i):
            l, k = i // nblk, i % nblk
            pltpu.async_copy(
                t128_hbm.at[idx_v.at[l, pl.ds(k * BBLK, BBLK)]],
                g_v.at[p], gsems[p])

        def wait_gather(p):
            pltpu.make_async_copy(
                t128_hbm.at[idx_v.at[0, pl.ds(0, BBLK)]],
                g_v.at[p], gsems[p]).wait()

        def fire_write(p, i):
            l, k = i // nblk, i % nblk
            pltpu.async_copy(
                g_v.at[p],
                out_hbm.at[l, pl.ds(b0 + k * BBLK, BBLK), :], wsems[p])

        def wait_write(p):
            pltpu.make_async_copy(
                g_v.at[p],
                out_hbm.at[0, pl.ds(0, BBLK), :], wsems[p]).wait()

        def outer(c, carry):
            for p in range(NBUF):
                @pl.when(c > 0)
                def _(p=p):
                    wait_write(p)

                fire_gather(p, c * NBUF + p)
            for p in range(NBUF):
                wait_gather(p)
                fire_write(p, c * NBUF + p)
            return carry

        lax.fori_loop(0, nblocks // NBUF, outer, 0, unroll=False)
        for p in range(NBUF):
            wait_write(p)

    return k1


@functools.lru_cache(maxsize=None)
def _build_transpose(emb, l_seq, batch):
    tblk = 1024

    def k2(in_ref, out_ref):
        rows = lax.broadcasted_iota(jnp.int32, (emb, emb), 0)
        cols = lax.broadcasted_iota(jnp.int32, (emb, emb), 1)
        eye = jnp.where(rows == cols, 1.0, 0.0).astype(jnp.float32)
        x = in_ref[0, :, :emb]
        out_ref[0] = lax.dot_general(
            eye, x, (((1,), (1,)), ((), ())),
            preferred_element_type=jnp.float32)

    return pl.pallas_call(
        k2,
        grid=(l_seq, batch // tblk),
        in_specs=[pl.BlockSpec((1, tblk, 128), lambda l, b: (l, b, 0))],
        out_specs=pl.BlockSpec((1, emb, tblk), lambda l, b: (l, 0, b)),
        out_shape=jax.ShapeDtypeStruct((l_seq, emb, batch), jnp.float32),
    )


def kernel(x, TokenEmbeddings):
    batch, l_seq = x.shape
    vocab, emb = TokenEmbeddings.shape
    t128 = jnp.pad(TokenEmbeddings, ((0, 0), (0, 128 - emb)))
    xt = x.T.astype(jnp.int32)
    mid = _build_gather(vocab, l_seq, batch)(t128, xt)
    out3 = _build_transpose(emb, l_seq, batch)(mid)
    return jnp.transpose(out3, (2, 0, 1))


# speedup vs baseline: 1.2603x; 1.1256x over previous
import functools

import jax
import jax.numpy as jnp
from jax import lax
from jax.experimental import pallas as pl
from jax.experimental.pallas import tpu as pltpu
from jax.experimental.pallas import tpu_sc as plsc

NC = 2
NS = 16
NW = NC * NS

IDXROW = 128
ROWS_PER_CHUNK = 256
NBUF = 4


@functools.lru_cache(maxsize=None)
def _build(vocab, emb, total):
    per_w = total // NW
    nrows = per_w // IDXROW
    chunks = per_w // ROWS_PER_CHUNK
    kpc = ROWS_PER_CHUNK // IDXROW
    n_outer = chunks // NBUF

    mesh = plsc.VectorSubcoreMesh(core_axis_name="c", subcore_axis_name="s")

    @functools.partial(
        pl.kernel,
        out_type=jax.ShapeDtypeStruct((total, emb), jnp.float32),
        mesh=mesh,
        scratch_types=[
            pltpu.VMEM((nrows, IDXROW), jnp.int32),
            pltpu.VMEM((NBUF, ROWS_PER_CHUNK, emb), jnp.float32),
            [pltpu.SemaphoreType.DMA] * NBUF,
            [pltpu.SemaphoreType.DMA] * NBUF,
        ],
        compiler_params=pltpu.CompilerParams(use_tc_tiling_on_sc=False),
    )
    def emb_kernel(table_hbm, idx_hbm, out_hbm, idx_v, rows_v, gsems, wsems):
        wid = lax.axis_index("s") * NC + lax.axis_index("c")
        base = wid * per_w

        pltpu.sync_copy(idx_hbm.at[wid], idx_v)

        def write_wait(b):
            pltpu.make_async_copy(
                rows_v.at[b],
                out_hbm.at[pl.ds(0, ROWS_PER_CHUNK)],
                wsems[b],
            ).wait()

        def outer(c0, carry):
            handles = []
            for b in range(NBUF):
                @pl.when(c0 > 0)
                def _(b=b):
                    write_wait(b)

                ch = c0 * NBUF + b
                hs = []
                for j in range(kpc):
                    hs.append(pltpu.async_copy(
                        table_hbm.at[idx_v.at[ch * kpc + j]],
                        rows_v.at[b, pl.ds(j * IDXROW, IDXROW)],
                        gsems[b],
                    ))
                handles.append(hs)
            for b in range(NBUF):
                for h in handles[b]:
                    h.wait()
                ch = c0 * NBUF + b
                pltpu.async_copy(
                    rows_v.at[b],
                    out_hbm.at[pl.ds(base + ch * ROWS_PER_CHUNK,
                                     ROWS_PER_CHUNK)],
                    wsems[b],
                )
            return carry

        lax.fori_loop(0, n_outer, outer, 0, unroll=False)
        for b in range(NBUF):
            write_wait(b)

    return emb_kernel


def kernel(x, TokenEmbeddings):
    b, l = x.shape
    vocab, emb = TokenEmbeddings.shape
    total = b * l
    idx = x.reshape(total).astype(jnp.int32)
    idx3 = idx.reshape(NW, total // NW // IDXROW, IDXROW)
    out = _build(vocab, emb, total)(TokenEmbeddings, idx3)
    return out.reshape(b, l, emb)
